# baseline (speedup 1.0000x reference)
import functools

import jax
import jax.numpy as jnp
from jax import lax
from jax.experimental import pallas as pl
from jax.experimental.pallas import tpu as pltpu
from jax.experimental.pallas import tpu_sc as plsc


_DIM = 32
_BATCH = 16384


@functools.lru_cache(maxsize=None)
def _build(V, D, B):
    info = plsc.get_sparse_core_info()
    NW = info.num_cores * info.num_subcores
    assert B % NW == 0
    b_per_w = B // NW
    n_batches = b_per_w // 16
    mesh = plsc.VectorSubcoreMesh(core_axis_name="c", subcore_axis_name="s")

    @functools.partial(
        pl.kernel,
        mesh=mesh,
        out_type=jax.ShapeDtypeStruct((B, D), jnp.float32),
        scratch_types=[
            pltpu.VMEM((b_per_w,), jnp.int32),
            pltpu.SemaphoreType.DMA,
        ],
    )
    def gather_kernel(idx_hbm, table_hbm, out_hbm, t_v, sem):
        wid = lax.axis_index("s") * info.num_cores + lax.axis_index("c")
        base = wid * b_per_w
        pltpu.sync_copy(idx_hbm.at[pl.ds(base, b_per_w)], t_v)

        pending = []
        for k in range(n_batches):
            tv = t_v[pl.ds(k * 16, 16)]
            issued = []
            for j in range(16):
                r = tv[j]
                issued.append(
                    pltpu.async_copy(
                        table_hbm.at[pl.ds(r, 1), :],
                        out_hbm.at[pl.ds(base + k * 16 + j, 1), :],
                        sem,
                    )
                )
            for c in pending:
                c.wait()
            pending = issued
        for c in pending:
            c.wait()

    return gather_kernel


def kernel(t, emb_weight):
    fn = _build(emb_weight.shape[0], _DIM, _BATCH)
    return fn(t.astype(jnp.int32), emb_weight)


# speedup vs baseline: 1.7731x; 1.7731x over previous
import functools

import jax
import jax.numpy as jnp
from jax import lax
from jax.experimental import pallas as pl
from jax.experimental.pallas import tpu as pltpu
from jax.experimental.pallas import tpu_sc as plsc


_DIM = 32
_BATCH = 16384


@functools.lru_cache(maxsize=None)
def _build(V, D, B):
    info = plsc.get_sparse_core_info()
    NW = info.num_cores * info.num_subcores
    assert B % NW == 0
    b_per_w = B // NW
    mesh = plsc.VectorSubcoreMesh(core_axis_name="c", subcore_axis_name="s")

    @functools.partial(
        pl.kernel,
        mesh=mesh,
        out_type=jax.ShapeDtypeStruct((B, D), jnp.float32),
        scratch_types=[
            pltpu.VMEM((b_per_w,), jnp.int32),
            pltpu.VMEM((b_per_w, D), jnp.float32),
            pltpu.SemaphoreType.DMA,
        ],
    )
    def gather_kernel(idx_hbm, table_hbm, out_hbm, t_v, rows_v, sem):
        wid = lax.axis_index("s") * info.num_cores + lax.axis_index("c")
        base = wid * b_per_w
        pltpu.sync_copy(idx_hbm.at[pl.ds(base, b_per_w)], t_v)

        copies = []
        for k in range(b_per_w // 16):
            tv = t_v[pl.ds(k * 16, 16)]
            for j in range(16):
                copies.append(
                    pltpu.async_copy(
                        table_hbm.at[pl.ds(tv[j], 1), :],
                        rows_v.at[pl.ds(k * 16 + j, 1), :],
                        sem,
                    )
                )
        for c in copies:
            c.wait()
        pltpu.sync_copy(rows_v, out_hbm.at[pl.ds(base, b_per_w)])

    return gather_kernel


def kernel(t, emb_weight):
    fn = _build(emb_weight.shape[0], _DIM, _BATCH)
    return fn(t.astype(jnp.int32), emb_weight)
